# Initial kernel scaffold; baseline (speedup 1.0000x reference)
import jax
import jax.numpy as jnp
from jax.experimental import pallas as pl


def kernel(hist_ids, wish_ids, bid, auth, lang, tags, dense, book_table, auth_table, lang_table, tag_table, dW0, db0, dW1, db1, dW2, db2, uW0, ub0, uW1, ub1, uW2, ub2):
    raise NotImplementedError("write your pallas kernel here")


# speedup vs baseline: 1.1287x; 1.1287x over previous
import functools

import jax
import jax.numpy as jnp
import numpy as np
from jax import lax
from jax.experimental import pallas as pl
from jax.experimental.pallas import tpu as pltpu
from jax.experimental.pallas import tpu_sc as plsc

B = 4096
E = 64
NC, NS = 2, 16
NW = NC * NS
BPW = B // NW
CHUNK = 128
N_BOOK_CHUNKS = BPW * (50 + 20 + 1) // CHUNK
N_TAG_CHUNKS = BPW * 10 // CHUNK
N_CHUNKS = N_BOOK_CHUNKS + 2 + N_TAG_CHUNKS
SLOT = 4 * BPW

_ar = np.arange(BPW)
_base_tgt = np.concatenate([
    np.repeat(_ar, 50),
    BPW + np.repeat(_ar, 20),
    3 * BPW + _ar,
    3 * BPW + _ar,
    3 * BPW + _ar,
    2 * BPW + np.repeat(_ar, 10),
]).astype(np.int32)
_TGT = (_base_tgt[None, :] + (np.arange(NS, dtype=np.int32) * SLOT)[:, None]
        ).reshape(NS, N_CHUNKS, CHUNK)


def _sc_gather_pool(book_t, auth_t, lang_t, tag_t, src_idx, tgt_idx, zeros):
    mesh = plsc.VectorSubcoreMesh(core_axis_name="c", subcore_axis_name="s")
    out_t = tuple(jax.ShapeDtypeStruct((B, E), jnp.float32) for _ in range(4))

    @functools.partial(
        pl.kernel,
        mesh=mesh,
        out_type=out_t,
        compiler_params=pltpu.CompilerParams(use_tc_tiling_on_sc=False),
        scratch_types=[
            pltpu.VMEM((N_CHUNKS, CHUNK), jnp.int32),
            pltpu.VMEM((N_CHUNKS, CHUNK), jnp.int32),
            pltpu.VMEM((CHUNK, E), jnp.float32),
            pltpu.VMEM_SHARED((NS * SLOT, E), jnp.float32),
        ],
    )
    def k(book_h, auth_h, lang_h, tag_h, srci_h, tgti_h, zero_h,
          oh_h, ow_h, ot_h, ob_h, srci_v, tgti_v, buf_v, acc_sh):
        c = lax.axis_index("c")
        s = lax.axis_index("s")
        wid = s * NC + c
        pltpu.sync_copy(srci_h.at[wid], srci_v)
        pltpu.sync_copy(tgti_h.at[s], tgti_v)
        pltpu.sync_copy(zero_h, acc_sh.at[pl.ds(s * SLOT, SLOT)])

        def do_chunk(table_h, j):
            pltpu.sync_copy(table_h.at[srci_v.at[j]], buf_v)
            pltpu.sync_copy(buf_v, acc_sh.at[tgti_v.at[j]], add=True)

        @pl.loop(0, N_BOOK_CHUNKS)
        def _(j):
            do_chunk(book_h, j)

        do_chunk(auth_h, N_BOOK_CHUNKS)
        do_chunk(lang_h, N_BOOK_CHUNKS + 1)

        @pl.loop(N_BOOK_CHUNKS + 2, N_CHUNKS)
        def _(j):
            do_chunk(tag_h, j)

        base = s * SLOT
        obase = wid * BPW
        pltpu.sync_copy(acc_sh.at[pl.ds(base, BPW)], oh_h.at[pl.ds(obase, BPW)])
        pltpu.sync_copy(acc_sh.at[pl.ds(base + BPW, BPW)], ow_h.at[pl.ds(obase, BPW)])
        pltpu.sync_copy(acc_sh.at[pl.ds(base + 2 * BPW, BPW)], ot_h.at[pl.ds(obase, BPW)])
        pltpu.sync_copy(acc_sh.at[pl.ds(base + 3 * BPW, BPW)], ob_h.at[pl.ds(obase, BPW)])

    return k(book_t, auth_t, lang_t, tag_t, src_idx, tgt_idx, zeros)


def _tc_combine(hist_s, wish_s, tag_s, bal_s, dense,
                dW0, db0, dW1, db1, dW2, db2, uW0, ub0, uW1, ub1, uW2, ub2):
    def body(hs, ws, ts, bs, dn, w0, b0, w1, b1, w2, b2,
             v0, c0, v1, c1, v2, c2, out):
        def dot_t(x, w_ref):
            return lax.dot_general(
                x, w_ref[...], (((1,), (1,)), ((), ())),
                preferred_element_type=jnp.float32,
                precision=lax.Precision.HIGHEST)

        u = hs[...] * (1.0 / 50.0) + ws[...] * (1.0 / 20.0)
        u = jnp.maximum(dot_t(u, v0) + c0[...], 0.0)
        u = jnp.maximum(dot_t(u, v1) + c1[...], 0.0)
        u = dot_t(u, v2) + c2[...]

        d = jnp.maximum(dot_t(dn[...], w0) + b0[...], 0.0)
        d = jnp.maximum(dot_t(d, w1) + b1[...], 0.0)
        d = dot_t(d, w2) + b2[...]

        i = bs[...] + ts[...] * (1.0 / 10.0) + d
        out[...] = jnp.sum(u * i, axis=1, keepdims=True)

    return pl.pallas_call(
        body,
        out_shape=jax.ShapeDtypeStruct((B, 1), jnp.float32),
    )(hist_s, wish_s, tag_s, bal_s, dense,
      dW0, db0.reshape(1, -1), dW1, db1.reshape(1, -1), dW2, db2.reshape(1, -1),
      uW0, ub0.reshape(1, -1), uW1, ub1.reshape(1, -1), uW2, ub2.reshape(1, -1))


def kernel(hist_ids, wish_ids, bid, auth, lang, tags, dense,
           book_table, auth_table, lang_table, tag_table,
           dW0, db0, dW1, db1, dW2, db2, uW0, ub0, uW1, ub1, uW2, ub2):
    i32 = lambda x: x.astype(jnp.int32)
    src_idx = jnp.concatenate([
        i32(hist_ids).reshape(NW, BPW * 50),
        i32(wish_ids).reshape(NW, BPW * 20),
        i32(bid).reshape(NW, BPW),
        i32(auth).reshape(NW, BPW),
        i32(lang).reshape(NW, BPW),
        i32(tags).reshape(NW, BPW * 10),
    ], axis=1).reshape(NW, N_CHUNKS, CHUNK)
    tgt_idx = jnp.asarray(_TGT)
    zeros = jnp.zeros((SLOT, E), jnp.float32)

    hist_s, wish_s, tag_s, bal_s = _sc_gather_pool(
        book_table, auth_table, lang_table, tag_table, src_idx, tgt_idx, zeros)

    return _tc_combine(hist_s, wish_s, tag_s, bal_s, dense,
                       dW0, db0, dW1, db1, dW2, db2,
                       uW0, ub0, uW1, ub1, uW2, ub2)
